# baseline (speedup 1.0000x reference)
import functools
import math

import jax
import jax.numpy as jnp
from jax import lax
from jax.experimental import pallas as pl
from jax.experimental.pallas import tpu as pltpu
from jax.experimental.pallas import tpu_sc as plsc

_LANES = 16
_CHUNK = 128


def _emb_kernel_body(n_per_w, n_ch, num_cores, scale, d_model, seq_len,
                     idx_hbm, table_hbm, out_hbm, idx_v, rows_v, *sems):
    gather_sems = sems[:n_ch]
    scatter_sem = sems[n_ch]
    wid = lax.axis_index("s") * num_cores + lax.axis_index("c")
    base = wid * n_per_w
    ob = base // seq_len
    os0 = base % seq_len
    pltpu.sync_copy(idx_hbm.at[wid], idx_v)
    gathers = [
        pltpu.async_copy(
            table_hbm.at[idx_v.at[j]],
            rows_v.at[pl.ds(j * _CHUNK, _CHUNK)],
            gather_sems[j],
        )
        for j in range(n_ch)
    ]
    scatters = []
    for j in range(n_ch):
        gathers[j].wait()

        @plsc.parallel_loop(j * _CHUNK, (j + 1) * _CHUNK, 1, unroll=4)
        def scale_row(i):
            for c in range(d_model // _LANES):
                sl = pl.ds(c * _LANES, _LANES)
                rows_v[i, sl] = rows_v[i, sl] * scale

        scatters.append(
            pltpu.async_copy(
                rows_v.at[pl.ds(j * _CHUNK, _CHUNK)],
                out_hbm.at[ob, pl.ds(os0 + j * _CHUNK, _CHUNK)],
                scatter_sem,
            )
        )
    for s in scatters:
        s.wait()


def kernel(indices, table):
    B, S = indices.shape
    V, D = table.shape
    N = B * S
    info = plsc.get_sparse_core_info()
    num_workers = info.num_cores * info.num_subcores
    n_per_w = N // num_workers
    n_ch = n_per_w // _CHUNK
    scale = jnp.float32(math.sqrt(float(D)))

    idx3 = indices.reshape(num_workers, n_ch, _CHUNK).astype(jnp.int32)
    mesh = plsc.VectorSubcoreMesh(core_axis_name="c", subcore_axis_name="s")

    k = functools.partial(
        pl.kernel,
        mesh=mesh,
        out_type=jax.ShapeDtypeStruct((B, S, D), jnp.float32),
        scratch_types=(
            [
                pltpu.VMEM((n_ch, _CHUNK), jnp.int32),
                pltpu.VMEM((n_per_w, D), jnp.float32),
            ]
            + [pltpu.SemaphoreType.DMA] * (n_ch + 1)
        ),
    )(functools.partial(_emb_kernel_body, n_per_w, n_ch, info.num_cores,
                        scale, D, S))

    return k(idx3, table)


# speedup vs baseline: 1.0133x; 1.0133x over previous
import functools
import math

import jax
import jax.numpy as jnp
from jax import lax
from jax.experimental import pallas as pl
from jax.experimental.pallas import tpu as pltpu
from jax.experimental.pallas import tpu_sc as plsc

_LANES = 16
_CHUNK = 64


def _emb_kernel_body(n_per_w, n_ch, num_cores, scale, d_model, seq_len,
                     idx_hbm, table_hbm, out_hbm, idx_v, rows_v, *sems):
    gather_sems = sems[:n_ch]
    scatter_sem = sems[n_ch]
    wid = lax.axis_index("s") * num_cores + lax.axis_index("c")
    base = wid * n_per_w
    ob = base // seq_len
    os0 = base % seq_len
    pltpu.sync_copy(idx_hbm.at[wid], idx_v)
    gathers = [
        pltpu.async_copy(
            table_hbm.at[idx_v.at[j]],
            rows_v.at[pl.ds(j * _CHUNK, _CHUNK)],
            gather_sems[j],
        )
        for j in range(n_ch)
    ]
    scatters = []
    for j in range(n_ch):
        gathers[j].wait()

        @plsc.parallel_loop(j * _CHUNK, (j + 1) * _CHUNK, 1, unroll=2)
        def scale_row(i):
            for c in range(d_model // _LANES):
                sl = pl.ds(c * _LANES, _LANES)
                rows_v[i, sl] = rows_v[i, sl] * scale

        scatters.append(
            pltpu.async_copy(
                rows_v.at[pl.ds(j * _CHUNK, _CHUNK)],
                out_hbm.at[ob, pl.ds(os0 + j * _CHUNK, _CHUNK)],
                scatter_sem,
            )
        )
    for s in scatters:
        s.wait()


def kernel(indices, table):
    B, S = indices.shape
    V, D = table.shape
    N = B * S
    info = plsc.get_sparse_core_info()
    num_workers = info.num_cores * info.num_subcores
    n_per_w = N // num_workers
    n_ch = n_per_w // _CHUNK
    scale = jnp.float32(math.sqrt(float(D)))

    idx3 = indices.reshape(num_workers, n_ch, _CHUNK).astype(jnp.int32)
    mesh = plsc.VectorSubcoreMesh(core_axis_name="c", subcore_axis_name="s")

    k = functools.partial(
        pl.kernel,
        mesh=mesh,
        out_type=jax.ShapeDtypeStruct((B, S, D), jnp.float32),
        scratch_types=(
            [
                pltpu.VMEM((n_ch, _CHUNK), jnp.int32),
                pltpu.VMEM((n_per_w, D), jnp.float32),
            ]
            + [pltpu.SemaphoreType.DMA] * (n_ch + 1)
        ),
    )(functools.partial(_emb_kernel_body, n_per_w, n_ch, info.num_cores,
                        scale, D, S))

    return k(idx3, table)
